# baseline (speedup 1.0000x reference)
import functools

import jax
import jax.numpy as jnp
from jax import lax
from jax.experimental import pallas as pl
from jax.experimental.pallas import tpu as pltpu

ALPHA_RELU = 0.15
IN_EPS = 1e-5


def _fused_kernel(x_ref, w_ref, o_ref, *, B, Cin, Cout, H, W):
    HW = H * W
    w = w_ref[...]

    col = lax.broadcasted_iota(jnp.int32, (1, HW), 1) % W
    mask_l = (col >= 1).astype(jnp.float32)
    mask_r = (col <= W - 2).astype(jnp.float32)

    for b in range(B):
        xb = x_ref[b].astype(jnp.bfloat16)
        zrow = jnp.zeros((Cin, W), jnp.bfloat16)
        z1 = jnp.zeros((Cout, 1), jnp.float32)
        p_stack = jnp.concatenate([
            jnp.concatenate([zrow, xb[:, :HW - W]], axis=1),
            xb,
            jnp.concatenate([xb[:, W:], zrow], axis=1),
        ], axis=0)

        y0 = jnp.dot(w[0], p_stack, preferred_element_type=jnp.float32)
        y1 = jnp.dot(w[1], p_stack, preferred_element_type=jnp.float32)
        y2 = jnp.dot(w[2], p_stack, preferred_element_type=jnp.float32)
        c0 = jnp.concatenate([z1, y0[:, :HW - 1]], axis=1) * mask_l
        c2 = jnp.concatenate([y2[:, 1:], z1], axis=1) * mask_r
        acc = y1 + c0 + c2

        inv_hw = jnp.float32(1.0 / HW)
        mean = jnp.sum(acc, axis=1, keepdims=True) * inv_hw
        ex2 = jnp.sum(acc * acc, axis=1, keepdims=True) * inv_hw
        var = ex2 - mean * mean
        s = lax.rsqrt(var + IN_EPS)
        normed = acc * s - mean * s

        out = jnp.maximum(normed, ALPHA_RELU * normed)
        o_ref[b] = out.astype(o_ref.dtype)


def _conv_layer_call(x_flat, w2, *, Cin, H, W, Cout, kh, kw):
    N = x_flat.shape[0]
    HW = H * W
    B = 8 if N % 8 == 0 else (4 if N % 4 == 0 else 1)
    Cout = w2.shape[1]
    kern = functools.partial(_fused_kernel, B=B, Cin=Cin, Cout=Cout, H=H, W=W)

    cost = pl.CostEstimate(
        flops=2 * N * HW * Cin * kh * kw * Cout,
        transcendentals=0,
        bytes_accessed=x_flat.size * 4 + w2.size * 2 + N * Cout * HW * 2,
    )

    return pl.pallas_call(
        kern,
        out_shape=jax.ShapeDtypeStruct((N, Cout, HW), jnp.bfloat16),
        grid=(N // B,),
        in_specs=[
            pl.BlockSpec((B, Cin, HW), lambda n: (n, 0, 0)),
            pl.BlockSpec((kw, Cout, kh * Cin), lambda n: (0, 0, 0)),
        ],
        out_specs=pl.BlockSpec((B, Cout, HW), lambda n: (n, 0, 0)),
        compiler_params=pltpu.CompilerParams(
            dimension_semantics=("parallel",),
            vmem_limit_bytes=64 * 1024 * 1024,
        ),
        cost_estimate=cost,
    )(x_flat, w2)


def kernel(x, weight, bias):
    del bias
    N, Cin, H, W = x.shape
    Cout, Cin_w, kh, kw = weight.shape
    assert Cin_w == Cin and kh == kw == 3
    HW = H * W

    x_flat = x.reshape(N, Cin, HW)
    w2 = jnp.transpose(weight, (3, 0, 2, 1)).reshape(kw, Cout, kh * Cin)
    w2 = w2.astype(jnp.bfloat16)

    out_flat = _conv_layer_call(x_flat, w2, Cin=Cin, H=H, W=W,
                                Cout=Cout, kh=kh, kw=kw)

    return out_flat.astype(jnp.float32).reshape(N, Cout, H, W)
